# baseline (speedup 1.0000x reference)
import functools

import jax
import jax.numpy as jnp
from jax import lax
from jax.experimental import pallas as pl
from jax.experimental.pallas import tpu as pltpu
from jax.experimental.pallas import tpu_sc as plsc

_B, _S, _D, _E, _K = 4, 4096, 2048, 16, 2
_M = _B * _S
_BM = 1024

_NC, _NSUB, _L = 2, 16, 16
_NW = _NC * _NSUB
_TPW = _M // _NW
_CH = _TPW // _L


def _matmul_body(x_ref, wt_hbm, logits_ref, wt_vmem, sem):
    @pl.when(pl.program_id(0) == 0)
    def _load_wt():
        cp = pltpu.make_async_copy(wt_hbm, wt_vmem, sem)
        cp.start()
        cp.wait()

    logits_ref[...] = jnp.dot(
        x_ref[...], wt_vmem[...], preferred_element_type=jnp.float32
    )


def _tc_logits(xm, wt):
    return pl.pallas_call(
        _matmul_body,
        grid=(_M // _BM,),
        in_specs=[
            pl.BlockSpec((_BM, _D), lambda i: (i, 0)),
            pl.BlockSpec(memory_space=pl.ANY),
        ],
        out_specs=pl.BlockSpec((_BM, _E), lambda i: (i, 0)),
        out_shape=jax.ShapeDtypeStruct((_M, _E), jnp.float32),
        scratch_shapes=[
            pltpu.VMEM((_D, _E), jnp.float32),
            pltpu.SemaphoreType.DMA,
        ],
    )(xm, wt)


def _sc_route_body(l_hbm, w_hbm, i_hbm, l_v, w_v, i_v):
    wid = lax.axis_index("s") * _NC + lax.axis_index("c")
    pltpu.sync_copy(l_hbm.at[pl.ds(wid * (_TPW * _E), _TPW * _E)], l_v)

    lanes = lax.iota(jnp.int32, _L)

    def chunk_body(c, carry):
        loc = c * _L + lanes
        m1 = jnp.full((_L,), -jnp.inf, jnp.float32)
        m2 = jnp.full((_L,), -jnp.inf, jnp.float32)
        i1 = jnp.zeros((_L,), jnp.int32)
        i2 = jnp.zeros((_L,), jnp.int32)
        for e in range(_E):
            ev = jnp.full((_L,), e, jnp.int32)
            v = plsc.load_gather(l_v, [loc * _E + e])
            gt1 = v > m1
            gt2 = jnp.logical_and(jnp.logical_not(gt1), v > m2)
            m2 = jnp.where(gt1, m1, jnp.where(gt2, v, m2))
            i2 = jnp.where(gt1, i1, jnp.where(gt2, ev, i2))
            m1 = jnp.where(gt1, v, m1)
            i1 = jnp.where(gt1, ev, i1)
        e2 = jnp.exp(m2 - m1)
        denom = 1.0 + e2
        plsc.store_scatter(w_v, [loc * _K], 1.0 / denom)
        plsc.store_scatter(w_v, [loc * _K + 1], e2 / denom)
        plsc.store_scatter(i_v, [loc * _K], i1)
        plsc.store_scatter(i_v, [loc * _K + 1], i2)
        return carry

    lax.fori_loop(0, _CH, chunk_body, 0)
    pltpu.sync_copy(w_v, w_hbm.at[pl.ds(wid * (_TPW * _K), _TPW * _K)])
    pltpu.sync_copy(i_v, i_hbm.at[pl.ds(wid * (_TPW * _K), _TPW * _K)])


def _sc_route(logits_flat):
    mesh = plsc.VectorSubcoreMesh(core_axis_name="c", subcore_axis_name="s")
    fn = functools.partial(
        pl.kernel,
        mesh=mesh,
        out_type=[
            jax.ShapeDtypeStruct((_M * _K,), jnp.float32),
            jax.ShapeDtypeStruct((_M * _K,), jnp.int32),
        ],
        scratch_types=[
            pltpu.VMEM((_TPW * _E,), jnp.float32),
            pltpu.VMEM((_TPW * _K,), jnp.float32),
            pltpu.VMEM((_TPW * _K,), jnp.int32),
        ],
        compiler_params=pltpu.CompilerParams(needs_layout_passes=False),
    )(_sc_route_body)
    return fn(logits_flat)


@jax.jit
def kernel(x, W):
    xm = x.reshape(_M, _D)
    wt = W.T
    logits = _tc_logits(xm, wt)
    w_flat, i_flat = _sc_route(logits.reshape(_M * _E))
    return (
        w_flat.reshape(_B, _S, _K),
        i_flat.reshape(_B, _S, _K),
        logits.reshape(_B, _S, _E),
    )


# speedup vs baseline: 2.5519x; 2.5519x over previous
import jax
import jax.numpy as jnp
from jax.experimental import pallas as pl

_B, _S, _D, _E, _K = 4, 4096, 2048, 16, 2
_M = _B * _S
_BM = 1024
_SPB = _S // _BM


def _router_body(x_ref, w_ref, lt_ref, wt_ref, it_ref):
    logits_t = jax.lax.dot_general(
        w_ref[...],
        x_ref[...],
        ((( 1,), (1,)), ((), ())),
        preferred_element_type=jnp.float32,
    )
    lt_ref[...] = logits_t[None]

    m1 = jnp.max(logits_t, axis=0)
    i1 = jnp.argmax(logits_t, axis=0).astype(jnp.int32)
    row = jax.lax.broadcasted_iota(jnp.int32, logits_t.shape, 0)
    masked = jnp.where(row == i1[None, :], -jnp.inf, logits_t)
    m2 = jnp.max(masked, axis=0)
    i2 = jnp.argmax(masked, axis=0).astype(jnp.int32)

    e2 = jnp.exp(m2 - m1)
    denom = 1.0 + e2
    wt_ref[...] = jnp.stack([1.0 / denom, e2 / denom], axis=0)[None]
    it_ref[...] = jnp.stack([i1, i2], axis=0)[None]


@jax.jit
def kernel(x, W):
    xm = x.reshape(_M, _D)

    lt, wt, it = pl.pallas_call(
        _router_body,
        grid=(_M // _BM,),
        in_specs=[
            pl.BlockSpec((_BM, _D), lambda i: (i, 0)),
            pl.BlockSpec((_E, _D), lambda i: (0, 0)),
        ],
        out_specs=[
            pl.BlockSpec((1, _E, _BM), lambda i: (i // _SPB, 0, i % _SPB)),
            pl.BlockSpec((1, _K, _BM), lambda i: (i // _SPB, 0, i % _SPB)),
            pl.BlockSpec((1, _K, _BM), lambda i: (i // _SPB, 0, i % _SPB)),
        ],
        out_shape=[
            jax.ShapeDtypeStruct((_B, _E, _S), jnp.float32),
            jax.ShapeDtypeStruct((_B, _K, _S), jnp.float32),
            jax.ShapeDtypeStruct((_B, _K, _S), jnp.int32),
        ],
    )(xm, W)

    return (
        wt.transpose(0, 2, 1),
        it.transpose(0, 2, 1),
        lt.transpose(0, 2, 1),
    )
